# baseline (speedup 1.0000x reference)
import jax
import jax.numpy as jnp
from jax.experimental import pallas as pl
from jax.experimental.pallas import tpu as pltpu

_INPUT = 2
_HID = 64
_VMEM_LIMIT = 56 * 1024 * 1024
_ROW_BLK = 512
_L1_BLK = 128


def _rowsum_encoder_body(a_ref, x_ref, w1_ref, b1_ref, w2_ref, b2_ref,
                         deg_ref, h_ref):
    deg_ref[...] = jnp.sum(a_ref[...], axis=1, keepdims=True)
    x = x_ref[...]
    h1 = x[:, 0:1] * w1_ref[0:1, :] + x[:, 1:2] * w1_ref[1:2, :] + b1_ref[...]
    h1 = jnp.maximum(h1, 0.0)
    h2 = jnp.dot(h1.astype(jnp.bfloat16), w2_ref[...],
                 preferred_element_type=jnp.float32) + b2_ref[...]
    h_ref[...] = h2.astype(h_ref.dtype)


def _gcn1_norm_body(a_ref, dinv_blk_ref, dinv_row_ref, h_ref, hblk_ref,
                    w_ref, b_ref, anorm_ref, out_ref):
    anorm = (a_ref[...] * dinv_blk_ref[...] * dinv_row_ref[...]
             ).astype(jnp.bfloat16)
    anorm_ref[...] = anorm
    agg = jnp.dot(anorm, h_ref[...], preferred_element_type=jnp.float32)
    agg += (dinv_blk_ref[...] * dinv_blk_ref[...]) * hblk_ref[...].astype(
        jnp.float32)
    out = jnp.dot(agg.astype(jnp.bfloat16), w_ref[...],
                  preferred_element_type=jnp.float32) + b_ref[...]
    out_ref[...] = out.astype(out_ref.dtype)


def _gcn_body(a_ref, h_ref, hblk_ref, dinv_ref, w_ref, b_ref, out_ref):
    agg = jnp.dot(a_ref[...], h_ref[...], preferred_element_type=jnp.float32)
    agg += (dinv_ref[...] * dinv_ref[...]) * hblk_ref[...].astype(jnp.float32)
    out = jnp.dot(agg.astype(jnp.bfloat16), w_ref[...],
                  preferred_element_type=jnp.float32) + b_ref[...]
    out_ref[...] = out.astype(out_ref.dtype)


def _gcn_decoder_body(a_ref, h_ref, hblk_ref, dinv_ref, w_ref, b_ref,
                      dw1_ref, db1_ref, dw2_ref, db2_ref, out_ref):
    agg = jnp.dot(a_ref[...], h_ref[...], preferred_element_type=jnp.float32)
    agg += (dinv_ref[...] * dinv_ref[...]) * hblk_ref[...].astype(jnp.float32)
    h3 = jnp.dot(agg.astype(jnp.bfloat16), w_ref[...],
                 preferred_element_type=jnp.float32) + b_ref[...]
    d = jnp.dot(h3.astype(jnp.bfloat16), dw1_ref[...],
                preferred_element_type=jnp.float32) + db1_ref[...]
    d = jnp.maximum(d, 0.0)
    o = jnp.sum(d * dw2_ref[...], axis=-1, keepdims=True) + db2_ref[...]
    out_ref[...] = jax.nn.sigmoid(o)


@jax.jit
def _forward(ew1, eb1, ew2, eb2, pw, pb, dw1, db1, dw2, db2,
             x, edge_index, edge_weight):
    n = x.shape[0]
    n_pad = ((n + _ROW_BLK - 1) // _ROW_BLK) * _ROW_BLK

    src = edge_index[0]
    dst = edge_index[1]
    a_raw = jnp.zeros((n_pad, n_pad), jnp.float32).at[dst, src].add(
        edge_weight, mode="promise_in_bounds")
    x_pad = jnp.zeros((n_pad, _INPUT), jnp.float32).at[:n].set(x)

    rowsum, h = pl.pallas_call(
        _rowsum_encoder_body,
        out_shape=(jax.ShapeDtypeStruct((n_pad, 1), jnp.float32),
                   jax.ShapeDtypeStruct((n_pad, _HID), jnp.bfloat16)),
        grid=(n_pad // _L1_BLK,),
        in_specs=[
            pl.BlockSpec((_L1_BLK, n_pad), lambda i: (i, 0)),
            pl.BlockSpec((_L1_BLK, _INPUT), lambda i: (i, 0)),
            pl.BlockSpec((_INPUT, _HID), lambda i: (0, 0)),
            pl.BlockSpec((1, _HID), lambda i: (0, 0)),
            pl.BlockSpec((_HID, _HID), lambda i: (0, 0)),
            pl.BlockSpec((1, _HID), lambda i: (0, 0)),
        ],
        out_specs=(pl.BlockSpec((_L1_BLK, 1), lambda i: (i, 0)),
                   pl.BlockSpec((_L1_BLK, _HID), lambda i: (i, 0))),
        compiler_params=pltpu.CompilerParams(
            dimension_semantics=("parallel",),
            vmem_limit_bytes=_VMEM_LIMIT),
    )(a_raw, x_pad, ew1, eb1, ew2.astype(jnp.bfloat16), eb2)

    node_mask = (jnp.arange(n_pad, dtype=jnp.int32) < n)[:, None]
    dinv_col = jnp.where(node_mask, jax.lax.rsqrt(rowsum + 1.0), 0.0)
    dinv_row = dinv_col.reshape(1, n_pad)

    a, h = pl.pallas_call(
        _gcn1_norm_body,
        out_shape=(jax.ShapeDtypeStruct((n_pad, n_pad), jnp.bfloat16),
                   jax.ShapeDtypeStruct((n_pad, _HID), jnp.bfloat16)),
        grid=(n_pad // _L1_BLK,),
        in_specs=[
            pl.BlockSpec((_L1_BLK, n_pad), lambda i: (i, 0)),
            pl.BlockSpec((_L1_BLK, 1), lambda i: (i, 0)),
            pl.BlockSpec((1, n_pad), lambda i: (0, 0)),
            pl.BlockSpec((n_pad, _HID), lambda i: (0, 0)),
            pl.BlockSpec((_L1_BLK, _HID), lambda i: (i, 0)),
            pl.BlockSpec((_HID, _HID), lambda i: (0, 0)),
            pl.BlockSpec((1, _HID), lambda i: (0, 0)),
        ],
        out_specs=(pl.BlockSpec((_L1_BLK, n_pad), lambda i: (i, 0)),
                   pl.BlockSpec((_L1_BLK, _HID), lambda i: (i, 0))),
        compiler_params=pltpu.CompilerParams(
            dimension_semantics=("parallel",),
            vmem_limit_bytes=_VMEM_LIMIT),
    )(a_raw, dinv_col, dinv_row, h, h, pw[0].astype(jnp.bfloat16), pb[0])

    grid = (n_pad // _ROW_BLK,)
    gcn_specs = [
        pl.BlockSpec((_ROW_BLK, n_pad), lambda i: (i, 0)),
        pl.BlockSpec((n_pad, _HID), lambda i: (0, 0)),
        pl.BlockSpec((_ROW_BLK, _HID), lambda i: (i, 0)),
        pl.BlockSpec((_ROW_BLK, 1), lambda i: (i, 0)),
        pl.BlockSpec((_HID, _HID), lambda i: (0, 0)),
        pl.BlockSpec((1, _HID), lambda i: (0, 0)),
    ]
    h = pl.pallas_call(
        _gcn_body,
        out_shape=jax.ShapeDtypeStruct((n_pad, _HID), jnp.bfloat16),
        grid=grid,
        in_specs=gcn_specs,
        out_specs=pl.BlockSpec((_ROW_BLK, _HID), lambda i: (i, 0)),
        compiler_params=pltpu.CompilerParams(
            dimension_semantics=("parallel",),
            vmem_limit_bytes=_VMEM_LIMIT),
    )(a, h, h, dinv_col, pw[1].astype(jnp.bfloat16), pb[1])

    out = pl.pallas_call(
        _gcn_decoder_body,
        out_shape=jax.ShapeDtypeStruct((n_pad, 1), jnp.float32),
        grid=grid,
        in_specs=gcn_specs + [
            pl.BlockSpec((_HID, _HID), lambda i: (0, 0)),
            pl.BlockSpec((1, _HID), lambda i: (0, 0)),
            pl.BlockSpec((1, _HID), lambda i: (0, 0)),
            pl.BlockSpec((1, 1), lambda i: (0, 0)),
        ],
        out_specs=pl.BlockSpec((_ROW_BLK, 1), lambda i: (i, 0)),
        compiler_params=pltpu.CompilerParams(
            dimension_semantics=("parallel",),
            vmem_limit_bytes=_VMEM_LIMIT),
    )(a, h, h, dinv_col, pw[2].astype(jnp.bfloat16), pb[2],
      dw1.astype(jnp.bfloat16), db1, dw2.T, db2)

    return out[:n]


def kernel(ew1, eb1, ew2, eb2, pw, pb, dw1, db1, dw2, db2,
           x, edge_index, edge_weight):
    return _forward(ew1, eb1, ew2, eb2, pw, pb, dw1, db1, dw2, db2,
                    x, edge_index, edge_weight)


# speedup vs baseline: 1.0179x; 1.0179x over previous
import jax
import jax.numpy as jnp
from jax.experimental import pallas as pl
from jax.experimental.pallas import tpu as pltpu

_INPUT = 2
_HID = 64
_VMEM_LIMIT = 56 * 1024 * 1024
_ROW_BLK = 512
_L1_BLK = 128


def _rowsum_encoder_body(a_ref, x_ref, w1_ref, b1_ref, w2_ref, b2_ref,
                         deg_ref, h_ref):
    deg_ref[...] = jnp.sum(a_ref[...], axis=1, keepdims=True)
    x = x_ref[...]
    h1 = x[:, 0:1] * w1_ref[0:1, :] + x[:, 1:2] * w1_ref[1:2, :] + b1_ref[...]
    h1 = jnp.maximum(h1, 0.0)
    h2 = jnp.dot(h1.astype(jnp.bfloat16), w2_ref[...],
                 preferred_element_type=jnp.float32) + b2_ref[...]
    h_ref[...] = h2.astype(h_ref.dtype)


def _gcn1_norm_body(a_ref, dinv_blk_ref, dinv_row_ref, h_ref, hblk_ref,
                    w_ref, b_ref, anorm_ref, out_ref):
    anorm = (a_ref[...] * dinv_blk_ref[...] * dinv_row_ref[...]
             ).astype(jnp.bfloat16)
    anorm_ref[...] = anorm
    agg = jnp.dot(anorm, h_ref[...], preferred_element_type=jnp.float32)
    agg += (dinv_blk_ref[...] * dinv_blk_ref[...]) * hblk_ref[...].astype(
        jnp.float32)
    out = jnp.dot(agg.astype(jnp.bfloat16), w_ref[...],
                  preferred_element_type=jnp.float32) + b_ref[...]
    out_ref[...] = out.astype(out_ref.dtype)


def _gcn_body(a_ref, h_ref, hblk_ref, dinv_ref, w_ref, b_ref, out_ref):
    agg = jnp.dot(a_ref[...], h_ref[...], preferred_element_type=jnp.float32)
    agg += (dinv_ref[...] * dinv_ref[...]) * hblk_ref[...].astype(jnp.float32)
    out = jnp.dot(agg.astype(jnp.bfloat16), w_ref[...],
                  preferred_element_type=jnp.float32) + b_ref[...]
    out_ref[...] = out.astype(out_ref.dtype)


def _gcn_decoder_body(a_ref, h_ref, hblk_ref, dinv_ref, w_ref, b_ref,
                      dw1_ref, db1_ref, dw2_ref, db2_ref, out_ref):
    agg = jnp.dot(a_ref[...], h_ref[...], preferred_element_type=jnp.float32)
    agg += (dinv_ref[...] * dinv_ref[...]) * hblk_ref[...].astype(jnp.float32)
    h3 = jnp.dot(agg.astype(jnp.bfloat16), w_ref[...],
                 preferred_element_type=jnp.float32) + b_ref[...]
    d = jnp.dot(h3.astype(jnp.bfloat16), dw1_ref[...],
                preferred_element_type=jnp.float32) + db1_ref[...]
    d = jnp.maximum(d, 0.0)
    o = jnp.sum(d * dw2_ref[...], axis=-1, keepdims=True) + db2_ref[...]
    out_ref[...] = jax.nn.sigmoid(o)


@jax.jit
def _forward(ew1, eb1, ew2, eb2, pw, pb, dw1, db1, dw2, db2,
             x, edge_index, edge_weight):
    n = x.shape[0]
    n_pad = ((n + _ROW_BLK - 1) // _ROW_BLK) * _ROW_BLK

    src = edge_index[0]
    dst = edge_index[1]
    lin = dst * n_pad + src
    a_raw = jnp.zeros((n_pad * n_pad,), jnp.float32).at[lin].add(
        edge_weight, mode="promise_in_bounds").reshape(n_pad, n_pad)
    x_pad = jnp.zeros((n_pad, _INPUT), jnp.float32).at[:n].set(x)

    rowsum, h = pl.pallas_call(
        _rowsum_encoder_body,
        out_shape=(jax.ShapeDtypeStruct((n_pad, 1), jnp.float32),
                   jax.ShapeDtypeStruct((n_pad, _HID), jnp.bfloat16)),
        grid=(n_pad // _L1_BLK,),
        in_specs=[
            pl.BlockSpec((_L1_BLK, n_pad), lambda i: (i, 0)),
            pl.BlockSpec((_L1_BLK, _INPUT), lambda i: (i, 0)),
            pl.BlockSpec((_INPUT, _HID), lambda i: (0, 0)),
            pl.BlockSpec((1, _HID), lambda i: (0, 0)),
            pl.BlockSpec((_HID, _HID), lambda i: (0, 0)),
            pl.BlockSpec((1, _HID), lambda i: (0, 0)),
        ],
        out_specs=(pl.BlockSpec((_L1_BLK, 1), lambda i: (i, 0)),
                   pl.BlockSpec((_L1_BLK, _HID), lambda i: (i, 0))),
        compiler_params=pltpu.CompilerParams(
            dimension_semantics=("parallel",),
            vmem_limit_bytes=_VMEM_LIMIT),
    )(a_raw, x_pad, ew1, eb1, ew2.astype(jnp.bfloat16), eb2)

    node_mask = (jnp.arange(n_pad, dtype=jnp.int32) < n)[:, None]
    dinv_col = jnp.where(node_mask, jax.lax.rsqrt(rowsum + 1.0), 0.0)
    dinv_row = dinv_col.reshape(1, n_pad)

    a, h = pl.pallas_call(
        _gcn1_norm_body,
        out_shape=(jax.ShapeDtypeStruct((n_pad, n_pad), jnp.bfloat16),
                   jax.ShapeDtypeStruct((n_pad, _HID), jnp.bfloat16)),
        grid=(n_pad // _L1_BLK,),
        in_specs=[
            pl.BlockSpec((_L1_BLK, n_pad), lambda i: (i, 0)),
            pl.BlockSpec((_L1_BLK, 1), lambda i: (i, 0)),
            pl.BlockSpec((1, n_pad), lambda i: (0, 0)),
            pl.BlockSpec((n_pad, _HID), lambda i: (0, 0)),
            pl.BlockSpec((_L1_BLK, _HID), lambda i: (i, 0)),
            pl.BlockSpec((_HID, _HID), lambda i: (0, 0)),
            pl.BlockSpec((1, _HID), lambda i: (0, 0)),
        ],
        out_specs=(pl.BlockSpec((_L1_BLK, n_pad), lambda i: (i, 0)),
                   pl.BlockSpec((_L1_BLK, _HID), lambda i: (i, 0))),
        compiler_params=pltpu.CompilerParams(
            dimension_semantics=("parallel",),
            vmem_limit_bytes=_VMEM_LIMIT),
    )(a_raw, dinv_col, dinv_row, h, h, pw[0].astype(jnp.bfloat16), pb[0])

    grid = (n_pad // _ROW_BLK,)
    gcn_specs = [
        pl.BlockSpec((_ROW_BLK, n_pad), lambda i: (i, 0)),
        pl.BlockSpec((n_pad, _HID), lambda i: (0, 0)),
        pl.BlockSpec((_ROW_BLK, _HID), lambda i: (i, 0)),
        pl.BlockSpec((_ROW_BLK, 1), lambda i: (i, 0)),
        pl.BlockSpec((_HID, _HID), lambda i: (0, 0)),
        pl.BlockSpec((1, _HID), lambda i: (0, 0)),
    ]
    h = pl.pallas_call(
        _gcn_body,
        out_shape=jax.ShapeDtypeStruct((n_pad, _HID), jnp.bfloat16),
        grid=grid,
        in_specs=gcn_specs,
        out_specs=pl.BlockSpec((_ROW_BLK, _HID), lambda i: (i, 0)),
        compiler_params=pltpu.CompilerParams(
            dimension_semantics=("parallel",),
            vmem_limit_bytes=_VMEM_LIMIT),
    )(a, h, h, dinv_col, pw[1].astype(jnp.bfloat16), pb[1])

    out = pl.pallas_call(
        _gcn_decoder_body,
        out_shape=jax.ShapeDtypeStruct((n_pad, 1), jnp.float32),
        grid=grid,
        in_specs=gcn_specs + [
            pl.BlockSpec((_HID, _HID), lambda i: (0, 0)),
            pl.BlockSpec((1, _HID), lambda i: (0, 0)),
            pl.BlockSpec((1, _HID), lambda i: (0, 0)),
            pl.BlockSpec((1, 1), lambda i: (0, 0)),
        ],
        out_specs=pl.BlockSpec((_ROW_BLK, 1), lambda i: (i, 0)),
        compiler_params=pltpu.CompilerParams(
            dimension_semantics=("parallel",),
            vmem_limit_bytes=_VMEM_LIMIT),
    )(a, h, h, dinv_col, pw[2].astype(jnp.bfloat16), pb[2],
      dw1.astype(jnp.bfloat16), db1, dw2.T, db2)

    return out[:n]


def kernel(ew1, eb1, ew2, eb2, pw, pb, dw1, db1, dw2, db2,
           x, edge_index, edge_weight):
    return _forward(ew1, eb1, ew2, eb2, pw, pb, dw1, db1, dw2, db2,
                    x, edge_index, edge_weight)
